# baseline (speedup 1.0000x reference)
import functools

import jax
import jax.numpy as jnp
from jax import lax
from jax.experimental import pallas as pl
from jax.experimental.pallas import tpu as pltpu
from jax.experimental.pallas import tpu_sc as plsc

B, T, V = 32, 16, 100000
N = B * T
NS = 16
ROWS = N // NS
L = 16
CHUNKS = ROWS // L


_mesh = plsc.VectorSubcoreMesh(core_axis_name="c", subcore_axis_name="s")

_SCRATCH = [
    pltpu.VMEM((ROWS,), jnp.int32),
    pltpu.VMEM((ROWS,), jnp.float32),
    pltpu.VMEM((ROWS,), jnp.int32),
    pltpu.VMEM((ROWS,), jnp.float32),
    pltpu.VMEM((ROWS,), jnp.float32),
    pltpu.VMEM((2, L), jnp.float32),
    pltpu.VMEM((L,), jnp.float32),
    pltpu.VMEM((NS, 2, L), jnp.float32),
    pltpu.HBM((NS, 2, L), jnp.float32),
    pltpu.SemaphoreType.DMA,
    pltpu.SemaphoreType.DMA,
]


def _nll_body(fine_hbm, final_hbm, tgt_hbm, msk_hbm, out_hbm,
                tgt_v, msk_v, idx_v, fine_v, final_v, stage_v, res_v, red_v,
                shared, sem_a, sem_b):
    cid = lax.axis_index("c")
    sid = lax.axis_index("s")

    @pl.when(cid == 0)
    def _work():
        base = pl.multiple_of(sid * ROWS, ROWS)
        pltpu.sync_copy(tgt_hbm.at[pl.ds(base, ROWS)], tgt_v)
        pltpu.sync_copy(msk_hbm.at[pl.ds(base, ROWS)], msk_v)

        lane = lax.iota(jnp.int32, L)
        for k in range(CHUNKS):
            rows = (base + k * L) + lane
            idx_v[pl.ds(k * L, L)] = rows * V + tgt_v[pl.ds(k * L, L)]

        cp_a = pltpu.async_copy(fine_hbm.at[idx_v], fine_v, sem_a)
        cp_b = pltpu.async_copy(final_hbm.at[idx_v], final_v, sem_b)
        cp_a.wait()
        cp_b.wait()

        lacc = jnp.zeros((L,), jnp.float32)
        macc = jnp.zeros((L,), jnp.float32)
        for k in range(CHUNKS):
            m = msk_v[pl.ds(k * L, L)]
            lacc = lacc + (fine_v[pl.ds(k * L, L)] + final_v[pl.ds(k * L, L)]) * m
            macc = macc + m

        stage_v[0] = lacc
        stage_v[1] = macc
        pltpu.sync_copy(stage_v, shared.at[sid])
        plsc.subcore_barrier()

        @pl.when(sid == 0)
        def _finish():
            pltpu.sync_copy(shared, red_v)
            lsum = jnp.zeros((L,), jnp.float32)
            msum = jnp.zeros((L,), jnp.float32)
            for r in range(NS):
                lsum = lsum + red_v[r, 0]
                msum = msum + red_v[r, 1]
            ltot = jnp.float32(0.0)
            mtot = jnp.float32(0.0)
            for i in range(L):
                ltot = ltot + lsum[i]
                mtot = mtot + msum[i]
            res_v[...] = jnp.broadcast_to(-ltot, (L,)) / jnp.broadcast_to(mtot, (L,))
            pltpu.sync_copy(res_v, out_hbm)


_nll_kernel = functools.partial(
    pl.kernel,
    out_type=jax.ShapeDtypeStruct((L,), jnp.float32),
    mesh=_mesh,
    scratch_types=_SCRATCH,
)(_nll_body)


def kernel(input_fine, input_final, target, mask):
    fine_flat = input_fine.reshape(-1)
    final_flat = input_final.reshape(-1)
    tgt_flat = target.reshape(-1).astype(jnp.int32)
    msk_flat = mask.reshape(-1).astype(jnp.float32)
    out = _nll_kernel(fine_flat, final_flat, tgt_flat, msk_flat)
    return out[0]


# speedup vs baseline: 22.8437x; 22.8437x over previous
import functools

import jax
import jax.numpy as jnp
from jax import lax
from jax.experimental import pallas as pl
from jax.experimental.pallas import tpu as pltpu
from jax.experimental.pallas import tpu_sc as plsc

B, T, V = 32, 16, 100000
N = B * T
NS = 16
ROWS = N // NS
L = 16
CHUNKS = ROWS // L
SEG = 128


_mesh = plsc.VectorSubcoreMesh(core_axis_name="c", subcore_axis_name="s")

_SCRATCH = [
    pltpu.VMEM((ROWS,), jnp.int32),
    pltpu.VMEM((ROWS,), jnp.float32),
    pltpu.VMEM((ROWS, 8, SEG), jnp.float32),
    pltpu.VMEM((ROWS, 8, SEG), jnp.float32),
    pltpu.VMEM((2, L), jnp.float32),
    pltpu.VMEM((L,), jnp.float32),
    pltpu.VMEM((NS, 2, L), jnp.float32),
    pltpu.HBM((NS, 2, L), jnp.float32),
    pltpu.SemaphoreType.DMA,
    pltpu.SemaphoreType.DMA,
]


def _nll_body(fine_hbm, final_hbm, tgt_hbm, msk_hbm, out_hbm,
              tgt_v, msk_v, fine_seg, final_seg, stage_v, res_v, red_v,
              bounce_hbm, sem_a, sem_b):
    cid = lax.axis_index("c")
    sid = lax.axis_index("s")

    @pl.when(cid == 0)
    def _work():
        base = pl.multiple_of(sid * ROWS, ROWS)
        pltpu.sync_copy(tgt_hbm.at[pl.ds(base, ROWS)], tgt_v)
        pltpu.sync_copy(msk_hbm.at[pl.ds(base, ROWS)], msk_v)

        descs = []
        for k in range(CHUNKS):
            tchunk = tgt_v[pl.ds(k * L, L)]
            for j in range(L):
                r = k * L + j
                b_s = sid * (ROWS // T) + (r // T)
                t0 = (r % T) & ~7
                tsc = tchunk[j]
                c0 = pl.multiple_of((tsc >> 7) << 7, SEG)
                d1 = pltpu.make_async_copy(
                    fine_hbm.at[b_s, pl.ds(t0, 8), pl.ds(c0, SEG)],
                    fine_seg.at[r], sem_a)
                d2 = pltpu.make_async_copy(
                    final_hbm.at[b_s, pl.ds(t0, 8), pl.ds(c0, SEG)],
                    final_seg.at[r], sem_b)
                d1.start()
                d2.start()
                descs.append(d1)
                descs.append(d2)
        for d in descs:
            d.wait()

        lacc = jnp.zeros((L,), jnp.float32)
        macc = jnp.zeros((L,), jnp.float32)
        lane = lax.iota(jnp.int32, L)
        subl = lane & 7
        for k in range(CHUNKS):
            colv = tgt_v[pl.ds(k * L, L)] & 127
            rowv = lane + (k * L)
            fvals = plsc.load_gather(fine_seg, [rowv, subl, colv])
            gvals = plsc.load_gather(final_seg, [rowv, subl, colv])
            m = msk_v[pl.ds(k * L, L)]
            lacc = lacc + (fvals + gvals) * m
            macc = macc + m

        stage_v[0] = lacc
        stage_v[1] = macc
        pltpu.sync_copy(stage_v, bounce_hbm.at[sid])
        plsc.subcore_barrier()

        @pl.when(sid == 0)
        def _finish():
            pltpu.sync_copy(bounce_hbm, red_v)
            lsum = jnp.zeros((L,), jnp.float32)
            msum = jnp.zeros((L,), jnp.float32)
            for r in range(NS):
                lsum = lsum + red_v[r, 0]
                msum = msum + red_v[r, 1]
            ltot = jnp.float32(0.0)
            mtot = jnp.float32(0.0)
            for i in range(L):
                ltot = ltot + lsum[i]
                mtot = mtot + msum[i]
            res_v[...] = jnp.broadcast_to(-ltot, (L,)) / jnp.broadcast_to(mtot, (L,))
            pltpu.sync_copy(res_v, out_hbm)


_nll_kernel = functools.partial(
    pl.kernel,
    out_type=jax.ShapeDtypeStruct((L,), jnp.float32),
    mesh=_mesh,
    scratch_types=_SCRATCH,
    compiler_params=pltpu.CompilerParams(needs_layout_passes=False),
)(_nll_body)


def kernel(input_fine, input_final, target, mask):
    tgt_flat = target.reshape(-1).astype(jnp.int32)
    msk_flat = mask.reshape(-1).astype(jnp.float32)
    out = _nll_kernel(input_fine, input_final, tgt_flat, msk_flat)
    return out[0]
